# baseline (speedup 1.0000x reference)
import jax
import jax.numpy as jnp
from jax import lax
from jax.experimental import pallas as pl
from jax.experimental.pallas import tpu as pltpu
from jax.experimental.pallas import tpu_sc as plsc

N_USERS = 5000
N_ITEMS = 5000
N = N_USERS + N_ITEMS
EMB = 128
HALF = EMB // 2
LAYERS = 3
E = 320000

NC = 2
NS = 16
EPT = E // NS
CH = 400
NCHUNK = EPT // CH
RPT = N // NS
ZCH = 400


def _body(src_hbm, dst_hbm, w_hbm, user_hbm, item_hbm,
          x0f, y1f, y2f, y3f, x0h, h1, h2,
          acc, dbig, sb0, sb1, wb0, wb1, gb0, gb1,
          ebsem, is0, is1, ws0, ws1, ge0, ge1, se0, se1):
    c = lax.axis_index("c")
    s = lax.axis_index("s")
    r0 = s * RPT
    c0 = c * HALF
    sbufs, wbufs, gbufs = (sb0, sb1), (wb0, wb1), (gb0, gb1)
    isems, wsems = (is0, is1), (ws0, ws1)
    gsems, ssems = (ge0, ge1), (se0, se1)

    zeros16 = jnp.zeros((16,), jnp.float32)

    def fill_zeros(i, carry):
        for cb in range(HALF // 16):
            gb0[i, pl.ds(cb * 16, 16)] = zeros16
        return carry

    def zero_acc_rows():
        lax.fori_loop(0, ZCH, fill_zeros, 0)
        done = 0
        while done < RPT:
            step = min(ZCH, RPT - done)
            pltpu.sync_copy(gb0.at[pl.ds(0, step)],
                            acc.at[pl.ds(r0 + done, step)])
            done += step

    pltpu.async_copy(dst_hbm.at[s], dbig, ebsem)

    def stage(emb, roff):
        done = 0
        while done < RPT:
            step = min(ZCH, RPT - done)
            pltpu.sync_copy(
                emb.at[pl.ds(roff + done, step), pl.ds(c0, HALF)],
                gb1.at[pl.ds(0, step)])
            pltpu.sync_copy(gb1.at[pl.ds(0, step)],
                            x0h.at[c, pl.ds(r0 + done, step)])
            pltpu.sync_copy(gb1.at[pl.ds(0, step)],
                            x0f.at[pl.ds(r0 + done, step), pl.ds(c0, HALF)])
            done += step

    @pl.when(s < NS // 2)
    def _():
        stage(user_hbm, r0)

    @pl.when(s >= NS // 2)
    def _():
        stage(item_hbm, r0 - N_USERS)

    zero_acc_rows()
    pltpu.make_async_copy(dst_hbm.at[0], dbig, ebsem).wait()
    plsc.subcore_barrier()

    def layer(xin, yfull, hout):

        def issue_src(kk, b):
            pltpu.async_copy(src_hbm.at[s, kk], sbufs[b], isems[b])

        def issue_w(kk, b):
            pltpu.async_copy(w_hbm.at[s, kk], wbufs[b], wsems[b])

        def wait_src(b):
            pltpu.make_async_copy(src_hbm.at[0, 0], sbufs[b],
                                  isems[b]).wait()

        def wait_w(b):
            pltpu.make_async_copy(w_hbm.at[0, 0], wbufs[b], wsems[b]).wait()

        def issue_gather(b):
            pltpu.async_copy(xin.at[sbufs[b]], gbufs[b], gsems[b])

        def wait_gather(b):
            pltpu.make_async_copy(xin.at[sbufs[b]], gbufs[b],
                                  gsems[b]).wait()

        def issue_scatter(kk, b):
            pltpu.async_copy(gbufs[b], acc.at[dbig.at[kk]], ssems[b],
                             add=True)

        def wait_scatter(b):
            pltpu.make_async_copy(gbufs[b], acc.at[dbig.at[0]],
                                  ssems[b]).wait()

        issue_src(0, 0)
        issue_w(0, 0)
        issue_src(1, 1)
        issue_w(1, 1)
        wait_src(0)
        issue_gather(0)

        def pair(k, carry):
            for b in range(2):
                kk = k + b
                wait_gather(b)

                @pl.when(kk >= 1)
                def _():
                    wait_scatter(1 - b)

                @pl.when(kk + 1 < NCHUNK)
                def _():
                    wait_src(1 - b)
                    issue_gather(1 - b)

                gbuf = gbufs[b]
                wbuf = wbufs[b]
                wait_w(b)

                def scale(g, inner):
                    wvec = wbuf[pl.ds(g * 16, 16)]
                    for j in range(16):
                        e = g * 16 + j
                        wj = wvec.at[jnp.full((16,), j, jnp.int32)].get(
                            mode="promise_in_bounds")
                        for cb in range(HALF // 16):
                            sl = pl.ds(cb * 16, 16)
                            gbuf[e, sl] = gbuf[e, sl] * wj
                    return inner

                lax.fori_loop(0, CH // 16, scale, 0, unroll=2)
                issue_scatter(kk, b)

                @pl.when(kk + 2 < NCHUNK)
                def _():
                    issue_src(kk + 2, b)
                    issue_w(kk + 2, b)
            return carry

        lax.fori_loop(0, NCHUNK // 2, lambda i, cy: pair(i * 2, cy), 0)
        wait_scatter((NCHUNK - 1) % 2)
        plsc.subcore_barrier()

        done = 0
        while done < RPT:
            step = min(ZCH, RPT - done)
            rows = pl.ds(r0 + done, step)
            if hout is not None:
                pltpu.sync_copy(acc.at[rows], hout.at[c, rows])
            pltpu.sync_copy(acc.at[rows],
                            yfull.at[rows, pl.ds(c0, HALF)])
            done += step
        if hout is not None:
            zero_acc_rows()
        plsc.subcore_barrier()

    layer(x0h.at[c], y1f, h1)
    layer(h1.at[c], y2f, h2)
    layer(h2.at[c], y3f, None)


@jax.jit
def _propagate(src, dst, w, user_emb, item_emb):
    f32 = jnp.float32
    out_type = [
        jax.ShapeDtypeStruct((N, EMB), f32),
        jax.ShapeDtypeStruct((N, EMB), f32),
        jax.ShapeDtypeStruct((N, EMB), f32),
        jax.ShapeDtypeStruct((N, EMB), f32),
        jax.ShapeDtypeStruct((NC, N, HALF), f32),
        jax.ShapeDtypeStruct((NC, N, HALF), f32),
        jax.ShapeDtypeStruct((NC, N, HALF), f32),
    ]
    run = pl.kernel(
        _body,
        out_type=out_type,
        mesh=plsc.VectorSubcoreMesh(core_axis_name="c", subcore_axis_name="s"),
        scratch_types=[
            pltpu.VMEM_SHARED((N, HALF), f32),
            pltpu.VMEM((NCHUNK, CH), jnp.int32),
            pltpu.VMEM((CH,), jnp.int32),
            pltpu.VMEM((CH,), jnp.int32),
            pltpu.VMEM((CH,), f32),
            pltpu.VMEM((CH,), f32),
            pltpu.VMEM((CH, HALF), f32),
            pltpu.VMEM((CH, HALF), f32),
            pltpu.SemaphoreType.DMA,
            pltpu.SemaphoreType.DMA,
            pltpu.SemaphoreType.DMA,
            pltpu.SemaphoreType.DMA,
            pltpu.SemaphoreType.DMA,
            pltpu.SemaphoreType.DMA,
            pltpu.SemaphoreType.DMA,
            pltpu.SemaphoreType.DMA,
            pltpu.SemaphoreType.DMA,
        ],
        compiler_params=pltpu.CompilerParams(use_tc_tiling_on_sc=False),
    )
    return run(src, dst, w, user_emb, item_emb)


def kernel(edge_index, edge_weight, user_emb, item_emb):
    src = edge_index[0].astype(jnp.int32).reshape(NS, NCHUNK, CH)
    dst = edge_index[1].astype(jnp.int32).reshape(NS, NCHUNK, CH)
    w = edge_weight.astype(jnp.float32).reshape(NS, NCHUNK, CH)
    outs = _propagate(src, dst, w, user_emb.astype(jnp.float32),
                      item_emb.astype(jnp.float32))
    return tuple(outs[:4])


# speedup vs baseline: 1.0308x; 1.0308x over previous
import jax
import jax.numpy as jnp
from jax import lax
from jax.experimental import pallas as pl
from jax.experimental.pallas import tpu as pltpu
from jax.experimental.pallas import tpu_sc as plsc

N_USERS = 5000
N_ITEMS = 5000
N = N_USERS + N_ITEMS
EMB = 128
HALF = EMB // 2
LAYERS = 3
E = 320000

NC = 2
NS = 16
EPT = E // NS
CH = 400
NCHUNK = EPT // CH
NB = 3
RPT = N // NS
ZCH = 400


def _body(src_hbm, dst_hbm, w_hbm, user_hbm, item_hbm,
          x0f, y1f, y2f, y3f, x0h, h1, h2,
          acc, sb0, sb1, sb2, db0, db1, db2, wb0, wb1, wb2,
          gb0, gb1, gb2,
          is0, is1, is2, ds0, ds1, ds2, ws0, ws1, ws2,
          ge0, ge1, ge2, se0, se1, se2):
    c = lax.axis_index("c")
    s = lax.axis_index("s")
    r0 = s * RPT
    c0 = c * HALF
    sbufs, dbufs, wbufs = (sb0, sb1, sb2), (db0, db1, db2), (wb0, wb1, wb2)
    gbufs = (gb0, gb1, gb2)
    isems, dsems, wsems = (is0, is1, is2), (ds0, ds1, ds2), (ws0, ws1, ws2)
    gsems, ssems = (ge0, ge1, ge2), (se0, se1, se2)

    zeros16 = jnp.zeros((16,), jnp.float32)

    def fill_zeros(i, carry):
        for cb in range(HALF // 16):
            gb0[i, pl.ds(cb * 16, 16)] = zeros16
        return carry

    def zero_acc_rows():
        lax.fori_loop(0, ZCH, fill_zeros, 0)
        done = 0
        while done < RPT:
            step = min(ZCH, RPT - done)
            pltpu.sync_copy(gb0.at[pl.ds(0, step)],
                            acc.at[pl.ds(r0 + done, step)])
            done += step

    def stage(emb, roff):
        done = 0
        while done < RPT:
            step = min(ZCH, RPT - done)
            pltpu.sync_copy(
                emb.at[pl.ds(roff + done, step), pl.ds(c0, HALF)],
                gb1.at[pl.ds(0, step)])
            pltpu.sync_copy(gb1.at[pl.ds(0, step)],
                            x0h.at[c, pl.ds(r0 + done, step)])
            pltpu.sync_copy(gb1.at[pl.ds(0, step)],
                            x0f.at[pl.ds(r0 + done, step), pl.ds(c0, HALF)])
            done += step

    @pl.when(s < NS // 2)
    def _():
        stage(user_hbm, r0)

    @pl.when(s >= NS // 2)
    def _():
        stage(item_hbm, r0 - N_USERS)

    zero_acc_rows()
    plsc.subcore_barrier()

    def layer(xin, yfull, hout):

        def issue_src(kk, m):
            pltpu.async_copy(src_hbm.at[s, kk], sbufs[m], isems[m])

        def issue_dst(kk, m):
            pltpu.async_copy(dst_hbm.at[s, kk], dbufs[m], dsems[m])

        def issue_w(kk, m):
            pltpu.async_copy(w_hbm.at[s, kk], wbufs[m], wsems[m])

        def wait_src(m):
            pltpu.make_async_copy(src_hbm.at[0, 0], sbufs[m],
                                  isems[m]).wait()

        def wait_dst(m):
            pltpu.make_async_copy(dst_hbm.at[0, 0], dbufs[m],
                                  dsems[m]).wait()

        def wait_w(m):
            pltpu.make_async_copy(w_hbm.at[0, 0], wbufs[m], wsems[m]).wait()

        def issue_gather(m):
            pltpu.async_copy(xin.at[sbufs[m]], gbufs[m], gsems[m])

        def wait_gather(m):
            pltpu.make_async_copy(xin.at[sbufs[m]], gbufs[m],
                                  gsems[m]).wait()

        def issue_scatter(m):
            pltpu.async_copy(gbufs[m], acc.at[dbufs[m]], ssems[m], add=True)

        def wait_scatter(m):
            pltpu.make_async_copy(gbufs[m], acc.at[dbufs[m]],
                                  ssems[m]).wait()

        for m in range(NB):
            issue_src(m, m)
            issue_w(m, m)
            issue_dst(m, m)
        wait_src(0)
        issue_gather(0)
        wait_src(1)
        issue_gather(1)

        def chunk(kk, m):
            @pl.when(kk < NCHUNK)
            def _():
                wait_gather(m)

                @pl.when(kk + 3 < NCHUNK)
                def _():
                    issue_src(kk + 3, m)

                p = (m + 2) % NB

                @pl.when(kk >= 1)
                def _():
                    wait_scatter(p)

                    @pl.when(kk + 2 < NCHUNK)
                    def _():
                        issue_dst(kk + 2, p)

                @pl.when(kk + 2 < NCHUNK)
                def _():
                    wait_src(p)
                    issue_gather(p)

                gbuf = gbufs[m]
                wbuf = wbufs[m]
                wait_w(m)

                def scale(g, inner):
                    wvec = wbuf[pl.ds(g * 16, 16)]
                    for j in range(16):
                        e = g * 16 + j
                        wj = wvec.at[jnp.full((16,), j, jnp.int32)].get(
                            mode="promise_in_bounds")
                        for cb in range(HALF // 16):
                            sl = pl.ds(cb * 16, 16)
                            gbuf[e, sl] = gbuf[e, sl] * wj
                    return inner

                lax.fori_loop(0, CH // 16, scale, 0, unroll=2)
                wait_dst(m)
                issue_scatter(m)

                @pl.when(kk + 3 < NCHUNK)
                def _():
                    issue_w(kk + 3, m)

        def triple(t, carry):
            for m in range(NB):
                chunk(t * NB + m, m)
            return carry

        ntrip = (NCHUNK + NB - 1) // NB
        lax.fori_loop(0, ntrip, triple, 0)
        wait_scatter((NCHUNK - 1) % NB)
        plsc.subcore_barrier()

        done = 0
        while done < RPT:
            step = min(ZCH, RPT - done)
            rows = pl.ds(r0 + done, step)
            if hout is not None:
                pltpu.sync_copy(acc.at[rows], hout.at[c, rows])
            pltpu.sync_copy(acc.at[rows],
                            yfull.at[rows, pl.ds(c0, HALF)])
            done += step
        if hout is not None:
            zero_acc_rows()
        plsc.subcore_barrier()

    layer(x0h.at[c], y1f, h1)
    layer(h1.at[c], y2f, h2)
    layer(h2.at[c], y3f, None)


@jax.jit
def _propagate(src, dst, w, user_emb, item_emb):
    f32 = jnp.float32
    i32 = jnp.int32
    out_type = [
        jax.ShapeDtypeStruct((N, EMB), f32),
        jax.ShapeDtypeStruct((N, EMB), f32),
        jax.ShapeDtypeStruct((N, EMB), f32),
        jax.ShapeDtypeStruct((N, EMB), f32),
        jax.ShapeDtypeStruct((NC, N, HALF), f32),
        jax.ShapeDtypeStruct((NC, N, HALF), f32),
        jax.ShapeDtypeStruct((NC, N, HALF), f32),
    ]
    scratch = [pltpu.VMEM_SHARED((N, HALF), f32)]
    scratch += [pltpu.VMEM((CH,), i32) for _ in range(2 * NB)]
    scratch += [pltpu.VMEM((CH,), f32) for _ in range(NB)]
    scratch += [pltpu.VMEM((CH, HALF), f32) for _ in range(NB)]
    scratch += [pltpu.SemaphoreType.DMA for _ in range(5 * NB)]
    run = pl.kernel(
        _body,
        out_type=out_type,
        mesh=plsc.VectorSubcoreMesh(core_axis_name="c", subcore_axis_name="s"),
        scratch_types=scratch,
        compiler_params=pltpu.CompilerParams(use_tc_tiling_on_sc=False),
    )
    return run(src, dst, w, user_emb, item_emb)


def kernel(edge_index, edge_weight, user_emb, item_emb):
    src = edge_index[0].astype(jnp.int32).reshape(NS, NCHUNK, CH)
    dst = edge_index[1].astype(jnp.int32).reshape(NS, NCHUNK, CH)
    w = edge_weight.astype(jnp.float32).reshape(NS, NCHUNK, CH)
    outs = _propagate(src, dst, w, user_emb.astype(jnp.float32),
                      item_emb.astype(jnp.float32))
    return tuple(outs[:4])


# speedup vs baseline: 1.0617x; 1.0299x over previous
import jax
import jax.numpy as jnp
from jax import lax
from jax.experimental import pallas as pl
from jax.experimental.pallas import tpu as pltpu
from jax.experimental.pallas import tpu_sc as plsc

N_USERS = 5000
N_ITEMS = 5000
N = N_USERS + N_ITEMS
EMB = 128
HALF = EMB // 2
LAYERS = 3
E = 320000

NC = 2
NS = 16
EPT = E // NS
CH = 400
NCHUNK = EPT // CH
NB = 3
RPT = N // NS
ZCH = 400


def _body(src_hbm, dst_hbm, w_hbm, user_hbm, item_hbm,
          x0f, y1f, y2f, y3f, x0h, h1, h2,
          acc, sb0, sb1, sb2, db0, db1, db2, wb0, wb1, wb2,
          gb0, gb1, gb2,
          is0, is1, is2, ds0, ds1, ds2, ws0, ws1, ws2,
          ge0, ge1, ge2, se0, se1, se2):
    c = lax.axis_index("c")
    s = lax.axis_index("s")
    r0 = s * RPT
    c0 = c * HALF
    sbufs, dbufs, wbufs = (sb0, sb1, sb2), (db0, db1, db2), (wb0, wb1, wb2)
    gbufs = (gb0, gb1, gb2)
    isems, dsems, wsems = (is0, is1, is2), (ds0, ds1, ds2), (ws0, ws1, ws2)
    gsems, ssems = (ge0, ge1, ge2), (se0, se1, se2)

    zeros16 = jnp.zeros((16,), jnp.float32)

    def fill_zeros(i, carry):
        for cb in range(HALF // 16):
            gb0[i, pl.ds(cb * 16, 16)] = zeros16
        return carry

    def zero_acc_rows():
        lax.fori_loop(0, ZCH, fill_zeros, 0)
        done = 0
        while done < RPT:
            step = min(ZCH, RPT - done)
            pltpu.sync_copy(gb0.at[pl.ds(0, step)],
                            acc.at[pl.ds(r0 + done, step)])
            done += step

    def stage(emb, roff):
        done = 0
        while done < RPT:
            step = min(ZCH, RPT - done)
            pltpu.sync_copy(
                emb.at[pl.ds(roff + done, step), pl.ds(c0, HALF)],
                gb1.at[pl.ds(0, step)])
            pltpu.sync_copy(gb1.at[pl.ds(0, step)],
                            x0h.at[c, pl.ds(r0 + done, step)])
            pltpu.sync_copy(gb1.at[pl.ds(0, step)],
                            x0f.at[pl.ds(r0 + done, step), pl.ds(c0, HALF)])
            done += step

    @pl.when(s < NS // 2)
    def _():
        stage(user_hbm, r0)

    @pl.when(s >= NS // 2)
    def _():
        stage(item_hbm, r0 - N_USERS)

    zero_acc_rows()
    plsc.subcore_barrier()

    def layer(xin, yfull, hout):

        def issue_src(kk, m):
            pltpu.async_copy(src_hbm.at[s, kk], sbufs[m], isems[m])

        def issue_dst(kk, m):
            pltpu.async_copy(dst_hbm.at[s, kk], dbufs[m], dsems[m])

        def issue_w(kk, m):
            pltpu.async_copy(w_hbm.at[s, kk], wbufs[m], wsems[m])

        def wait_src(m):
            pltpu.make_async_copy(src_hbm.at[0, 0], sbufs[m],
                                  isems[m]).wait()

        def wait_dst(m):
            pltpu.make_async_copy(dst_hbm.at[0, 0], dbufs[m],
                                  dsems[m]).wait()

        def wait_w(m):
            pltpu.make_async_copy(w_hbm.at[0, 0], wbufs[m], wsems[m]).wait()

        def issue_gather(m):
            pltpu.async_copy(xin.at[sbufs[m]], gbufs[m], gsems[m])

        def wait_gather(m):
            pltpu.make_async_copy(xin.at[sbufs[m]], gbufs[m],
                                  gsems[m]).wait()

        def issue_scatter(m):
            pltpu.async_copy(gbufs[m], acc.at[dbufs[m]], ssems[m], add=True)

        def wait_scatter(m):
            pltpu.make_async_copy(gbufs[m], acc.at[dbufs[m]],
                                  ssems[m]).wait()

        for m in range(NB):
            issue_src(m, m)
            issue_w(m, m)
            issue_dst(m, m)
        wait_src(0)
        issue_gather(0)
        wait_src(1)
        issue_gather(1)

        def chunk(kk, m):
            @pl.when(kk < NCHUNK)
            def _():
                wait_gather(m)

                @pl.when(kk + 3 < NCHUNK)
                def _():
                    issue_src(kk + 3, m)

                p = (m + 2) % NB

                gbuf = gbufs[m]
                wbuf = wbufs[m]
                wait_w(m)

                def scale(g, inner):
                    wvec = wbuf[pl.ds(g * 16, 16)]
                    for j in range(16):
                        e = g * 16 + j
                        wj = wvec.at[jnp.full((16,), j, jnp.int32)].get(
                            mode="promise_in_bounds")
                        for cb in range(HALF // 16):
                            sl = pl.ds(cb * 16, 16)
                            gbuf[e, sl] = gbuf[e, sl] * wj
                    return inner

                lax.fori_loop(0, CH // 16, scale, 0, unroll=2)

                @pl.when(kk >= 1)
                def _():
                    wait_scatter(p)

                    @pl.when(kk + 2 < NCHUNK)
                    def _():
                        issue_dst(kk + 2, p)

                @pl.when(kk + 2 < NCHUNK)
                def _():
                    wait_src(p)
                    issue_gather(p)

                wait_dst(m)
                issue_scatter(m)

                @pl.when(kk + 3 < NCHUNK)
                def _():
                    issue_w(kk + 3, m)

        def triple(t, carry):
            for m in range(NB):
                chunk(t * NB + m, m)
            return carry

        ntrip = (NCHUNK + NB - 1) // NB
        lax.fori_loop(0, ntrip, triple, 0)
        wait_scatter((NCHUNK - 1) % NB)
        plsc.subcore_barrier()

        done = 0
        while done < RPT:
            step = min(ZCH, RPT - done)
            rows = pl.ds(r0 + done, step)
            if hout is not None:
                pltpu.sync_copy(acc.at[rows], hout.at[c, rows])
            pltpu.sync_copy(acc.at[rows],
                            yfull.at[rows, pl.ds(c0, HALF)])
            done += step
        if hout is not None:
            zero_acc_rows()
        plsc.subcore_barrier()

    layer(x0h.at[c], y1f, h1)
    layer(h1.at[c], y2f, h2)
    layer(h2.at[c], y3f, None)


@jax.jit
def _propagate(src, dst, w, user_emb, item_emb):
    f32 = jnp.float32
    i32 = jnp.int32
    out_type = [
        jax.ShapeDtypeStruct((N, EMB), f32),
        jax.ShapeDtypeStruct((N, EMB), f32),
        jax.ShapeDtypeStruct((N, EMB), f32),
        jax.ShapeDtypeStruct((N, EMB), f32),
        jax.ShapeDtypeStruct((NC, N, HALF), f32),
        jax.ShapeDtypeStruct((NC, N, HALF), f32),
        jax.ShapeDtypeStruct((NC, N, HALF), f32),
    ]
    scratch = [pltpu.VMEM_SHARED((N, HALF), f32)]
    scratch += [pltpu.VMEM((CH,), i32) for _ in range(2 * NB)]
    scratch += [pltpu.VMEM((CH,), f32) for _ in range(NB)]
    scratch += [pltpu.VMEM((CH, HALF), f32) for _ in range(NB)]
    scratch += [pltpu.SemaphoreType.DMA for _ in range(5 * NB)]
    run = pl.kernel(
        _body,
        out_type=out_type,
        mesh=plsc.VectorSubcoreMesh(core_axis_name="c", subcore_axis_name="s"),
        scratch_types=scratch,
        compiler_params=pltpu.CompilerParams(use_tc_tiling_on_sc=False),
    )
    return run(src, dst, w, user_emb, item_emb)


def kernel(edge_index, edge_weight, user_emb, item_emb):
    src = edge_index[0].astype(jnp.int32).reshape(NS, NCHUNK, CH)
    dst = edge_index[1].astype(jnp.int32).reshape(NS, NCHUNK, CH)
    w = edge_weight.astype(jnp.float32).reshape(NS, NCHUNK, CH)
    outs = _propagate(src, dst, w, user_emb.astype(jnp.float32),
                      item_emb.astype(jnp.float32))
    return tuple(outs[:4])
